# baseline (speedup 1.0000x reference)
import jax
import jax.numpy as jnp
from jax import lax
from jax.experimental import pallas as pl
from jax.experimental.pallas import tpu as pltpu
from jax.experimental.pallas import tpu_sc as plsc
import functools

TOKENS = 100
DIM = 1024
BATCH = 1024
B = BATCH * TOKENS

NC, NS = 2, 16
NW = NC * NS
B_PER_W = B // NW
CHUNK = 16
K = 5
NCHUNKS = B_PER_W // CHUNK
NGROUPS = NCHUNKS // K


def _make_kernel():
    mesh = plsc.VectorSubcoreMesh(core_axis_name="c", subcore_axis_name="s")

    @functools.partial(
        pl.kernel,
        out_type=jax.ShapeDtypeStruct((B, DIM), jnp.float32),
        mesh=mesh,
        scratch_types=[
            pltpu.VMEM((B_PER_W,), jnp.int32),
            pltpu.VMEM((K, CHUNK, DIM), jnp.float32),
        ]
        + [pltpu.SemaphoreType.DMA] * (2 * K),
    )
    def emb(idx_hbm, table_hbm, out_hbm, idx_v, rows_v, *sems):
        gsems, ssems = sems[:K], sems[K:]
        wid = lax.axis_index("s") * NC + lax.axis_index("c")
        base = wid * B_PER_W
        pltpu.sync_copy(idx_hbm.at[pl.ds(base, B_PER_W)], idx_v)

        def gather(c, b):
            return pltpu.make_async_copy(
                table_hbm.at[idx_v.at[pl.ds(c * CHUNK, CHUNK)]],
                rows_v.at[b],
                gsems[b],
            )

        def scatter(c, b):
            return pltpu.make_async_copy(
                rows_v.at[b],
                out_hbm.at[pl.ds(base + c * CHUNK, CHUNK)],
                ssems[b],
            )

        for b in range(K):
            gather(b, b).start()

        def group(g, carry):
            for b in range(K):
                c = g * K + b
                gather(c, b).wait()
                scatter(c, b).start()
                scatter(c, b).wait()
                gather(c + K, b).start()
            return carry

        lax.fori_loop(0, NGROUPS - 1, group, 0)

        for b in range(K):
            c = (NGROUPS - 1) * K + b
            gather(c, b).wait()
            scatter(c, b).start()
        for b in range(K):
            c = (NGROUPS - 1) * K + b
            scatter(c, b).wait()

    return emb


_emb = _make_kernel()


@jax.jit
def kernel(indices, embedding_weight):
    idx_flat = indices.reshape(B).astype(jnp.int32)
    out = _emb(idx_flat, embedding_weight)
    return out.reshape(BATCH, TOKENS, DIM)


# speedup vs baseline: 1.5156x; 1.5156x over previous
import jax
import jax.numpy as jnp
from jax import lax
from jax.experimental import pallas as pl
from jax.experimental.pallas import tpu as pltpu
from jax.experimental.pallas import tpu_sc as plsc
import functools

TOKENS = 100
DIM = 1024
BATCH = 1024
B = BATCH * TOKENS

NC, NS = 2, 16
NW = NC * NS
B_PER_W = B // NW
L = 16
NSTEPS = B_PER_W // L


def _make_kernel():
    mesh = plsc.VectorSubcoreMesh(core_axis_name="c", subcore_axis_name="s")

    @functools.partial(
        pl.kernel,
        out_type=jax.ShapeDtypeStruct((B, DIM), jnp.float32),
        mesh=mesh,
        scratch_types=[
            pltpu.VMEM((TOKENS, DIM), jnp.float32),
            pltpu.VMEM((B_PER_W,), jnp.int32),
            pltpu.SemaphoreType.DMA,
        ],
    )
    def emb(idx_hbm, table_hbm, out_hbm, table_v, idx_v, ssem):
        wid = lax.axis_index("s") * NC + lax.axis_index("c")
        base = wid * B_PER_W
        pltpu.sync_copy(table_hbm, table_v)
        pltpu.sync_copy(idx_hbm.at[pl.ds(base, B_PER_W)], idx_v)

        def fire_step(ci, carry):
            coff = base + ci * L
            vec = idx_v[pl.ds(ci * L, L)]
            for j in range(L):
                i = jnp.squeeze(lax.slice(vec, (j,), (j + 1,)))
                pltpu.make_async_copy(
                    table_v.at[i], out_hbm.at[coff + j], ssem
                ).start()
            return carry

        lax.fori_loop(0, NSTEPS, fire_step, 0)

        def drain_step(r, carry):
            pltpu.make_async_copy(
                table_v.at[0], out_hbm.at[base + r], ssem
            ).wait()
            return carry

        lax.fori_loop(0, B_PER_W, drain_step, 0)

    return emb


_emb = _make_kernel()


@jax.jit
def kernel(indices, embedding_weight):
    idx_flat = indices.reshape(B).astype(jnp.int32)
    out = _emb(idx_flat, embedding_weight)
    return out.reshape(BATCH, TOKENS, DIM)


# speedup vs baseline: 2.7107x; 1.7885x over previous
import jax
import jax.numpy as jnp
from jax import lax
from jax.experimental import pallas as pl
from jax.experimental.pallas import tpu as pltpu
from jax.experimental.pallas import tpu_sc as plsc
import functools

TOKENS = 100
DIM = 1024
BATCH = 1024
B = BATCH * TOKENS

NC, NS = 2, 16
NW = NC * NS
B_PER_W = B // NW
L = 16
NSTEPS = B_PER_W // L


def _make_kernel():
    mesh = plsc.VectorSubcoreMesh(core_axis_name="c", subcore_axis_name="s")

    @functools.partial(
        pl.kernel,
        out_type=jax.ShapeDtypeStruct((BATCH, TOKENS, DIM), jnp.float32),
        mesh=mesh,
        scratch_types=[
            pltpu.VMEM((TOKENS, DIM), jnp.float32),
            pltpu.VMEM((B_PER_W,), jnp.int32),
            pltpu.SemaphoreType.DMA,
        ],
        compiler_params=pltpu.CompilerParams(use_tc_tiling_on_sc=True),
    )
    def emb(idx_hbm, table_hbm, out_hbm, table_v, idx_v, ssem):
        wid = lax.axis_index("s") * NC + lax.axis_index("c")
        base = wid * B_PER_W
        pltpu.sync_copy(table_hbm, table_v)
        pltpu.sync_copy(idx_hbm.at[pl.ds(base, B_PER_W)], idx_v)

        def fire_step(ci, carry):
            roff = base + ci * L
            vec = idx_v[pl.ds(ci * L, L)]
            for j in range(L):
                i = jnp.squeeze(lax.slice(vec, (j,), (j + 1,)))
                rr = roff + j
                pltpu.make_async_copy(
                    table_v.at[i], out_hbm.at[rr // TOKENS, rr % TOKENS], ssem
                ).start()
            return carry

        lax.fori_loop(0, NSTEPS, fire_step, 0)

        def drain_step(r, carry):
            pltpu.make_async_copy(
                table_v.at[0], out_hbm.at[0, 0], ssem
            ).wait()
            return carry

        lax.fori_loop(0, B_PER_W, drain_step, 0)

    return emb


_emb = _make_kernel()


@jax.jit
def kernel(indices, embedding_weight):
    idx_flat = indices.reshape(B).astype(jnp.int32)
    return _emb(idx_flat, embedding_weight)
